# baseline (speedup 1.0000x reference)
import functools

import numpy as np
import jax
import jax.numpy as jnp
from jax import lax
from jax.experimental import pallas as pl
from jax.experimental.pallas import tpu as pltpu
from jax.experimental.pallas import tpu_sc as plsc

VOCAB = 1000000
EMBED_DIM = 64
SEQ = 200
BATCH = 4096
SCALE = 8.0

LANES = 16
NUM_WORKERS = 32
ROWS_PER_WORKER = BATCH * SEQ // NUM_WORKERS
B_CHUNK = 8
CHUNK_ROWS = B_CHUNK * SEQ
NUM_CHUNKS = ROWS_PER_WORKER // CHUNK_ROWS


def _positional_encoder(seq_length, embed_dim):
    position = np.arange(seq_length, dtype=np.float32)[:, None]
    div_term = np.exp(
        np.arange(0, embed_dim, 2, dtype=np.float32)[None, :]
        * -(np.log(10000.0) / embed_dim))
    pe = np.zeros((seq_length, embed_dim), dtype=np.float32)
    pe[:, 0::2] = np.sin(position * div_term)
    pe[:, 1::2] = np.cos(position * div_term)
    return pe

_PE = _positional_encoder(SEQ, EMBED_DIM)


def _body(x_hbm, table_hbm, pe_hbm, out_hbm, idx_v, rows_v, pe_v, sem):
    wid = lax.axis_index("s") * 2 + lax.axis_index("c")
    base = wid * ROWS_PER_WORKER

    pltpu.sync_copy(pe_hbm, pe_v)

    def chunk_body(c, _):
        start = base + c * CHUNK_ROWS
        pltpu.sync_copy(x_hbm.at[pl.ds(start, CHUNK_ROWS)], idx_v)
        pltpu.async_copy(table_hbm.at[idx_v], rows_v, sem).wait()

        def pos_body(p, _):
            pe_regs = [pe_v[p, pl.ds(j * LANES, LANES)]
                       for j in range(EMBED_DIM // LANES)]
            for b in range(B_CHUNK):
                r = b * SEQ
                for j in range(EMBED_DIM // LANES):
                    sl = pl.ds(j * LANES, LANES)
                    rows_v[r + p, sl] = rows_v[r + p, sl] * SCALE + pe_regs[j]
            return ()

        lax.fori_loop(0, SEQ, pos_body, (), unroll=False)
        pltpu.sync_copy(rows_v, out_hbm.at[pl.ds(start, CHUNK_ROWS)])
        return ()

    lax.fori_loop(0, NUM_CHUNKS, chunk_body, (), unroll=False)


@functools.partial(jax.jit, donate_argnums=())
def kernel(x, table):
    x_flat = x.reshape(-1)
    mesh = plsc.VectorSubcoreMesh(core_axis_name="c", subcore_axis_name="s")
    run = pl.kernel(
        _body,
        mesh=mesh,
        out_type=jax.ShapeDtypeStruct((BATCH * SEQ, EMBED_DIM), jnp.float32),
        scratch_types=[
            pltpu.VMEM((CHUNK_ROWS,), jnp.int32),
            pltpu.VMEM((CHUNK_ROWS, EMBED_DIM), jnp.float32),
            pltpu.VMEM((SEQ, EMBED_DIM), jnp.float32),
            pltpu.SemaphoreType.DMA,
        ],
        compiler_params=pltpu.CompilerParams(use_tc_tiling_on_sc=False),
    )
    out = run(x_flat, table, jnp.asarray(_PE))
    return out.reshape(BATCH, SEQ, EMBED_DIM)
